# baseline (speedup 1.0000x reference)
import functools

import jax
import jax.numpy as jnp
from jax import lax
from jax.experimental import pallas as pl
from jax.experimental.pallas import tpu as pltpu
from jax.experimental.pallas import tpu_sc as plsc

HIDDEN = 64
BATCH = 16384
LMBDA = 0.0001
NC = 2
NS = 16
NW = NC * NS
LANES = 16
B2 = 2 * BATCH
PER_W = B2 // NW
CHUNK = 128
NCHUNK = PER_W // CHUNK
KH = HIDDEN // LANES


def _sc_kernel(h_hbm, t_hbm, r_hbm, ent_hbm, rel_hbm,
               q_hbm, sq_hbm,
               hidx_v, tidx_v, ridx_v,
               hrow_v, trow_v, rrow_v,
               qbuf_v, sq_v,
               sem0, sem1, sem2):
    wid = lax.axis_index("s") * NC + lax.axis_index("c")
    base = wid * PER_W

    pltpu.sync_copy(h_hbm.at[pl.ds(base, PER_W)], hidx_v)
    pltpu.sync_copy(t_hbm.at[pl.ds(base, PER_W)], tidx_v)
    pltpu.sync_copy(r_hbm.at[pl.ds(base, PER_W)], ridx_v)

    sq_acc = jnp.zeros((LANES,), jnp.float32)

    for c in range(NCHUNK):
        cp0 = pltpu.async_copy(
            ent_hbm.at[hidx_v.at[pl.ds(c * CHUNK, CHUNK)]], hrow_v, sem0)
        cp1 = pltpu.async_copy(
            ent_hbm.at[tidx_v.at[pl.ds(c * CHUNK, CHUNK)]], trow_v, sem1)
        cp2 = pltpu.async_copy(
            rel_hbm.at[ridx_v.at[pl.ds(c * CHUNK, CHUNK)]], rrow_v, sem2)
        cp0.wait()
        cp1.wait()
        cp2.wait()

        def item_body(i, sq):
            q = jnp.zeros((LANES,), jnp.float32)
            for k in range(KH):
                h = hrow_v[i, pl.ds(k * LANES, LANES)]
                t = trow_v[i, pl.ds(k * LANES, LANES)]
                r = rrow_v[i, pl.ds(k * LANES, LANES)]
                q = q + h * r * t
                sq = sq + h * h + t * t + r * r
            qbuf_v[i, :] = q
            return sq

        sq_acc = lax.fori_loop(0, CHUNK, item_body, sq_acc)

        pltpu.sync_copy(qbuf_v, q_hbm.at[pl.ds(base + c * CHUNK, CHUNK)])

    sq_v[...] = sq_acc
    pltpu.sync_copy(sq_v, sq_hbm.at[wid])


_sc_call = functools.partial(
    pl.kernel,
    out_type=(
        jax.ShapeDtypeStruct((B2, LANES), jnp.float32),
        jax.ShapeDtypeStruct((NW, LANES), jnp.float32),
    ),
    mesh=plsc.VectorSubcoreMesh(
        core_axis_name="c", subcore_axis_name="s",
        num_cores=NC, num_subcores=NS),
    scratch_types=[
        pltpu.VMEM((PER_W,), jnp.int32),
        pltpu.VMEM((PER_W,), jnp.int32),
        pltpu.VMEM((PER_W,), jnp.int32),
        pltpu.VMEM((CHUNK, HIDDEN), jnp.float32),
        pltpu.VMEM((CHUNK, HIDDEN), jnp.float32),
        pltpu.VMEM((CHUNK, HIDDEN), jnp.float32),
        pltpu.VMEM((CHUNK, LANES), jnp.float32),
        pltpu.VMEM((LANES,), jnp.float32),
        pltpu.SemaphoreType.DMA,
        pltpu.SemaphoreType.DMA,
        pltpu.SemaphoreType.DMA,
    ],
    compiler_params=pltpu.CompilerParams(use_tc_tiling_on_sc=False),
)(_sc_kernel)


def _tc_kernel(q_ref, y_ref, sq_ref, o_ref):
    row = lax.broadcasted_iota(jnp.int32, (128, 8), 0)
    col = lax.broadcasted_iota(jnp.int32, (128, 8), 1)
    fold = (row // LANES == col).astype(jnp.float32)
    scores = jnp.dot(q_ref[...], fold,
                     preferred_element_type=jnp.float32)
    x = -y_ref[...] * scores
    sp = jnp.maximum(x, 0.0) + jnp.log1p(jnp.exp(-jnp.abs(x)))
    loss = jnp.sum(sp) / BATCH
    regul = jnp.sum(sq_ref[...]) / (BATCH * HIDDEN)
    o_ref[0, 0] = loss + LMBDA * regul


_tc_call = pl.pallas_call(
    _tc_kernel,
    out_shape=jax.ShapeDtypeStruct((1, 1), jnp.float32),
    out_specs=pl.BlockSpec(memory_space=pltpu.SMEM),
)


def kernel(pos_h, pos_t, pos_r, neg_h, neg_t, neg_r, pos_y, neg_y,
           ent_emb, rel_emb):
    h_all = jnp.concatenate([pos_h, neg_h]).astype(jnp.int32)
    t_all = jnp.concatenate([pos_t, neg_t]).astype(jnp.int32)
    r_all = jnp.concatenate([pos_r, neg_r]).astype(jnp.int32)
    y_all = jnp.concatenate([pos_y, neg_y]).reshape(B2 // 8, 8)

    q, sq = _sc_call(h_all, t_all, r_all, ent_emb, rel_emb)
    loss = _tc_call(q.reshape(B2 // 8, 128), y_all,
                    sq.reshape(NW * LANES // 128, 128))
    return loss[0, 0]


# speedup vs baseline: 1.6177x; 1.6177x over previous
import functools

import jax
import jax.numpy as jnp
from jax import lax
from jax.experimental import pallas as pl
from jax.experimental.pallas import tpu as pltpu
from jax.experimental.pallas import tpu_sc as plsc

HIDDEN = 64
BATCH = 16384
LMBDA = 0.0001
NC = 2
NS = 16
NW = NC * NS
LANES = 16
B2 = 2 * BATCH
PER_W = B2 // NW
CHUNK = 256
NCHUNK = PER_W // CHUNK
KH = HIDDEN // LANES
GROUPS = CHUNK // 8


def _sc_kernel(h_hbm, t_hbm, r_hbm, ent_hbm, rel_hbm,
               q_hbm, sq_hbm,
               hidx_v, tidx_v, ridx_v,
               hrow_v, trow_v, rrow_v,
               qbuf_v, sq_v,
               sem0, sem1, sem2):
    wid = lax.axis_index("s") * NC + lax.axis_index("c")
    base = wid * PER_W

    pltpu.sync_copy(h_hbm.at[pl.ds(base, PER_W)], hidx_v)
    pltpu.sync_copy(t_hbm.at[pl.ds(base, PER_W)], tidx_v)
    pltpu.sync_copy(r_hbm.at[pl.ds(base, PER_W)], ridx_v)

    sq_acc = jnp.zeros((LANES,), jnp.float32)

    for c in range(NCHUNK):
        def fire_body(jj, _):
            hv = hidx_v[pl.ds(c * CHUNK + jj * LANES, LANES)]
            tv = tidx_v[pl.ds(c * CHUNK + jj * LANES, LANES)]
            rv = ridx_v[pl.ds(c * CHUNK + jj * LANES, LANES)]
            for u in range(LANES):
                j = jj * LANES + u
                pltpu.async_copy(ent_hbm.at[hv[u]], hrow_v.at[j], sem0)
                pltpu.async_copy(ent_hbm.at[tv[u]], trow_v.at[j], sem1)
                pltpu.async_copy(rel_hbm.at[rv[u]], rrow_v.at[j], sem2)
            return 0

        lax.fori_loop(0, CHUNK // LANES, fire_body, 0)
        dummy = ent_hbm.at[pl.ds(0, CHUNK)]
        pltpu.make_async_copy(dummy, hrow_v, sem0).wait()
        pltpu.make_async_copy(dummy, trow_v, sem1).wait()
        pltpu.make_async_copy(dummy, rrow_v, sem2).wait()

        def group_body(g, sq):
            for u in range(8):
                q = jnp.zeros((LANES,), jnp.float32)
                for k in range(KH):
                    h = hrow_v[g * 8 + u, pl.ds(k * LANES, LANES)]
                    t = trow_v[g * 8 + u, pl.ds(k * LANES, LANES)]
                    r = rrow_v[g * 8 + u, pl.ds(k * LANES, LANES)]
                    q = q + h * r * t
                    sq = sq + h * h + t * t + r * r
                qbuf_v[pl.ds(g * 128 + u * LANES, LANES)] = q
            return sq

        sq_acc = lax.fori_loop(0, GROUPS, group_body, sq_acc)

        pltpu.sync_copy(
            qbuf_v,
            q_hbm.at[pl.ds((base + c * CHUNK) * LANES, CHUNK * LANES)])

    sq_v[...] = sq_acc
    pltpu.sync_copy(sq_v, sq_hbm.at[pl.ds(wid * LANES, LANES)])


_sc_call = functools.partial(
    pl.kernel,
    out_type=(
        jax.ShapeDtypeStruct((B2 * LANES,), jnp.float32),
        jax.ShapeDtypeStruct((NW * LANES,), jnp.float32),
    ),
    mesh=plsc.VectorSubcoreMesh(
        core_axis_name="c", subcore_axis_name="s",
        num_cores=NC, num_subcores=NS),
    scratch_types=[
        pltpu.VMEM((PER_W,), jnp.int32),
        pltpu.VMEM((PER_W,), jnp.int32),
        pltpu.VMEM((PER_W,), jnp.int32),
        pltpu.VMEM((CHUNK, HIDDEN), jnp.float32),
        pltpu.VMEM((CHUNK, HIDDEN), jnp.float32),
        pltpu.VMEM((CHUNK, HIDDEN), jnp.float32),
        pltpu.VMEM((CHUNK * LANES,), jnp.float32),
        pltpu.VMEM((LANES,), jnp.float32),
        pltpu.SemaphoreType.DMA,
        pltpu.SemaphoreType.DMA,
        pltpu.SemaphoreType.DMA,
    ],
    compiler_params=pltpu.CompilerParams(use_tc_tiling_on_sc=True),
)(_sc_kernel)


def _tc_kernel(q_ref, y_ref, sq_ref, o_ref):
    row = lax.broadcasted_iota(jnp.int32, (128, 8), 0)
    col = lax.broadcasted_iota(jnp.int32, (128, 8), 1)
    fold = (row // LANES == col).astype(jnp.float32)
    scores = jnp.dot(q_ref[...], fold,
                     preferred_element_type=jnp.float32)
    x = -y_ref[...] * scores
    sp = jnp.maximum(x, 0.0) + jnp.log1p(jnp.exp(-jnp.abs(x)))
    loss = jnp.sum(sp) / BATCH
    regul = jnp.sum(sq_ref[...]) / (BATCH * HIDDEN)
    o_ref[0, 0] = loss + LMBDA * regul


_tc_call = pl.pallas_call(
    _tc_kernel,
    out_shape=jax.ShapeDtypeStruct((1, 1), jnp.float32),
    out_specs=pl.BlockSpec(memory_space=pltpu.SMEM),
)


def kernel(pos_h, pos_t, pos_r, neg_h, neg_t, neg_r, pos_y, neg_y,
           ent_emb, rel_emb):
    h_all = jnp.concatenate([pos_h, neg_h]).astype(jnp.int32)
    t_all = jnp.concatenate([pos_t, neg_t]).astype(jnp.int32)
    r_all = jnp.concatenate([pos_r, neg_r]).astype(jnp.int32)
    y_all = jnp.concatenate([pos_y, neg_y]).reshape(B2 // 8, 8)

    q, sq = _sc_call(h_all, t_all, r_all, ent_emb, rel_emb)
    loss = _tc_call(q.reshape(B2 // 8, 128), y_all,
                    sq.reshape(NW * LANES // 128, 128))
    return loss[0, 0]
